# Initial kernel scaffold; baseline (speedup 1.0000x reference)
import jax
import jax.numpy as jnp
from jax.experimental import pallas as pl


def kernel(x, edge_index, batch, W1, b1, W2, b2, W3, b3, W4, b4, c5w, c5b, c6w, c6b, f1w, f1b, f2w, f2b):
    raise NotImplementedError("write your pallas kernel here")


# speedup vs baseline: 6.0052x; 6.0052x over previous
import functools

import jax
import jax.numpy as jnp
from jax import lax
from jax.experimental import pallas as pl
from jax.experimental.pallas import tpu as pltpu
from jax.experimental.pallas import tpu_sc as plsc

_NC = 2
_NS = 16
_NW = _NC * _NS
_CH = 80
_NEG = -3.0e38


def _make_sc_scatter(n2, epad):
    epw = epad // _NW
    nchunks = epw // _CH
    rps = n2 // _NS
    mesh = plsc.VectorSubcoreMesh(
        core_axis_name="c", subcore_axis_name="s",
        num_cores=_NC, num_subcores=_NS)
    n_trash = n2 - 8

    def body(row_hbm, col_hbm, table_hbm, zeros_hbm, out_hbm,
             rowv, colv, col2v, rowsv, shared, sem):
        cid = lax.axis_index("c")
        sid = lax.axis_index("s")
        wid = sid * _NC + cid
        base = wid * epw
        pltpu.sync_copy(zeros_hbm.at[pl.ds(sid * rps, rps)],
                        shared.at[pl.ds(sid * rps, rps)])
        plsc.subcore_barrier()

        def chunk(i, carry):
            off = base + i * _CH
            pltpu.sync_copy(row_hbm.at[pl.ds(off, _CH)], rowv)
            pltpu.sync_copy(col_hbm.at[pl.ds(off, _CH)], colv)
            for q in range(_CH // 16):
                r = rowv[pl.ds(q * 16, 16)]
                c = colv[pl.ds(q * 16, 16)]
                col2v[pl.ds(q * 16, 16)] = jnp.where(r == c, n_trash, c)
            pltpu.async_copy(table_hbm.at[rowv], rowsv, sem).wait()
            pltpu.sync_copy(rowsv, shared.at[col2v], add=True)
            return carry

        lax.fori_loop(0, nchunks, chunk, 0)
        plsc.subcore_barrier()
        pltpu.sync_copy(shared.at[pl.ds(sid * rps, rps)],
                        out_hbm.at[cid, pl.ds(sid * rps, rps)])

    return pl.kernel(
        body,
        out_type=jax.ShapeDtypeStruct((_NC, n2, 32), jnp.float32),
        mesh=mesh,
        compiler_params=pltpu.CompilerParams(use_tc_tiling_on_sc=False),
        scratch_types=[
            pltpu.VMEM((_CH,), jnp.int32),
            pltpu.VMEM((_CH,), jnp.int32),
            pltpu.VMEM((_CH,), jnp.int32),
            pltpu.VMEM((_CH, 32), jnp.float32),
            pltpu.VMEM_SHARED((n2, 32), jnp.float32),
            pltpu.SemaphoreType.DMA,
        ],
    )


def _dis_body(p_ref, o_ref):
    deg = 1.0 + p_ref[0, :, 0:1] + p_ref[1, :, 0:1]
    o_ref[...] = lax.rsqrt(deg)


def _hd_body(x_ref, w_ref, dis_ref, o_ref):
    o_ref[...] = dis_ref[...] * jnp.dot(
        x_ref[...], w_ref[...], preferred_element_type=jnp.float32)


def _post_body(p_ref, hd_ref, dis_ref, b_ref, o_ref):
    s = p_ref[0] + p_ref[1] + hd_ref[...]
    o_ref[...] = jnp.tanh(dis_ref[...] * s + b_ref[...])


def _read_body(xc_ref, bt_ref, lv_ref, c5w_ref, c5b_ref, c6w_ref, c6b_ref,
               f1w_ref, f1b_ref, f2w_ref, f2b_ref, o_ref):
    g = pl.program_id(0)
    bt = bt_ref[...]
    mask = bt == g
    cnt = jnp.sum(mask.astype(jnp.int32))
    v = jnp.where(mask, lv_ref[...], _NEG)
    iota = lax.broadcasted_iota(jnp.int32, v.shape, 1)
    big = jnp.int32(2 ** 30)
    ohs = []
    for j in range(30):
        m = jnp.max(v)
        idx = jnp.min(jnp.where(v == m, iota, big))
        oh = iota == idx
        valid = jnp.where(j < cnt, 1.0, 0.0)
        ohs.append(oh.astype(jnp.float32) * valid)
        v = jnp.where(oh, _NEG, v)
    ohm = jnp.concatenate(ohs, axis=0)
    kept = jnp.dot(ohm, xc_ref[...], preferred_element_type=jnp.float32)
    h1 = jnp.maximum(
        jnp.dot(kept, c5w_ref[...], preferred_element_type=jnp.float32)
        + c5b_ref[...], 0.0)
    pool = jnp.concatenate(
        [jnp.maximum(h1[2 * t:2 * t + 1, :], h1[2 * t + 1:2 * t + 2, :])
         for t in range(15)], axis=0)
    acc = None
    for dt in range(5):
        part = jnp.dot(pool[dt:dt + 11, :], c6w_ref[dt],
                       preferred_element_type=jnp.float32)
        acc = part if acc is None else acc + part
    h2 = jnp.maximum(acc + c6b_ref[...], 0.0)
    f = None
    for t in range(11):
        ft = jnp.dot(h2[t:t + 1, :], f1w_ref[t],
                     preferred_element_type=jnp.float32)
        f = ft if f is None else f + ft
    f = jnp.maximum(f + f1b_ref[...], 0.0)
    lg = jnp.dot(f, f2w_ref[...], preferred_element_type=jnp.float32) \
        + f2b_ref[...]
    m2 = jnp.max(lg, axis=1, keepdims=True)
    lg = lg - m2
    ls = lg - jnp.log(jnp.sum(jnp.exp(lg), axis=1, keepdims=True))
    o_ref[...] = jnp.broadcast_to(ls[:, None, :], o_ref.shape)


def _full(shape):
    nd = len(shape)
    return pl.BlockSpec(shape, lambda g, _n=nd: (0,) * _n)


def kernel(x, edge_index, batch, W1, b1, W2, b2, W3, b3, W4, b4,
           c5w, c5b, c6w, c6b, f1w, f1b, f2w, f2b):
    N, F = x.shape
    E = edge_index.shape[1]
    B = 100
    C = f2b.shape[0]
    n2 = ((N + 16 + 127) // 128) * 128
    np3 = ((n2 + 127) // 128) * 128
    epad = ((E + _NW * _CH - 1) // (_NW * _CH)) * (_NW * _CH)

    row = edge_index[0]
    col = edge_index[1]
    if epad > E:
        zpad = jnp.zeros((epad - E,), row.dtype)
        row = jnp.concatenate([row, zpad])
        col = jnp.concatenate([col, zpad])
    row = row.astype(jnp.int32)
    col = col.astype(jnp.int32)

    zeros_tab = jnp.zeros((n2, 32), jnp.float32)
    ones_tab = jnp.ones((n2, 32), jnp.float32)
    sc = _make_sc_scatter(n2, epad)

    p_deg = sc(row, col, ones_tab, zeros_tab)
    dis = pl.pallas_call(
        _dis_body,
        out_shape=jax.ShapeDtypeStruct((n2, 1), jnp.float32),
    )(p_deg)

    def layer(h_in, W, b, width):
        hd = pl.pallas_call(
            _hd_body,
            out_shape=jax.ShapeDtypeStruct((n2, width), jnp.float32),
        )(h_in, W, dis)
        hd32 = hd if width == 32 else jnp.pad(hd, ((0, 0), (0, 32 - width)))
        p = sc(row, col, hd32, zeros_tab)
        pc = p if width == 32 else p[:, :, :width]
        return pl.pallas_call(
            _post_body,
            out_shape=jax.ShapeDtypeStruct((n2, width), jnp.float32),
        )(pc, hd, dis, b.reshape(1, width))

    xp = jnp.pad(x, ((0, n2 - N), (0, 0)))
    x1 = layer(xp, W1, b1, 32)
    x2 = layer(x1, W2, b2, 32)
    x3 = layer(x2, W3, b3, 32)
    x4 = layer(x3, W4, b4, 1)

    xc = jnp.concatenate([x1, x2, x3, x4], axis=-1)
    D = xc.shape[1]
    xcp = jnp.pad(xc, ((0, np3 - n2), (0, 128 - D)))
    lastv = jnp.pad(x4[:, 0], (0, np3 - n2),
                    constant_values=_NEG).reshape(1, np3)
    batchv = jnp.pad(batch.astype(jnp.int32), (0, np3 - N),
                     constant_values=B).reshape(1, np3)

    c5p = jnp.pad(c5w.reshape(16, D).T, ((0, 128 - D), (0, 0)))
    c6p = jnp.transpose(c6w, (2, 1, 0))
    f1p = jnp.transpose(f1w.reshape(32, 11, 128), (1, 0, 2))

    out = pl.pallas_call(
        _read_body,
        grid=(B,),
        in_specs=[
            _full((np3, 128)), _full((1, np3)), _full((1, np3)),
            _full((128, 16)), _full((1, 16)),
            _full((5, 16, 32)), _full((1, 32)),
            _full((11, 32, 128)), _full((1, 128)),
            _full((128, C)), _full((1, C)),
        ],
        out_specs=pl.BlockSpec((1, 8, C), lambda g: (g, 0, 0)),
        out_shape=jax.ShapeDtypeStruct((B, 8, C), jnp.float32),
    )(xcp, batchv, lastv, c5p, c5b.reshape(1, 16), c6p, c6b.reshape(1, 32),
      f1p, f1b.reshape(1, 128), f2w, f2b.reshape(1, C))
    return out[:, 0, :]


# speedup vs baseline: 6.4341x; 1.0714x over previous
import functools

import jax
import jax.numpy as jnp
from jax import lax
from jax.experimental import pallas as pl
from jax.experimental.pallas import tpu as pltpu
from jax.experimental.pallas import tpu_sc as plsc

_NC = 2
_NS = 16
_NW = _NC * _NS
_CH = 128
_NEG = -3.0e38


def _make_sc_scatter(n2, epad):
    epw = epad // _NW
    nchunks = epw // _CH
    rps = n2 // _NS
    mesh = plsc.VectorSubcoreMesh(
        core_axis_name="c", subcore_axis_name="s",
        num_cores=_NC, num_subcores=_NS)
    n_trash = n2 - 8

    def body(row_hbm, col_hbm, table_hbm, zeros_hbm, out_hbm,
             rowv, colv, col2v, rowsv, shared, sem):
        cid = lax.axis_index("c")
        sid = lax.axis_index("s")
        wid = sid * _NC + cid
        base = wid * epw
        pltpu.sync_copy(zeros_hbm.at[pl.ds(sid * rps, rps)],
                        shared.at[pl.ds(sid * rps, rps)])
        plsc.subcore_barrier()

        def chunk(i, carry):
            off = base + i * _CH
            pltpu.sync_copy(row_hbm.at[pl.ds(off, _CH)], rowv)
            pltpu.sync_copy(col_hbm.at[pl.ds(off, _CH)], colv)
            for q in range(_CH // 16):
                r = rowv[pl.ds(q * 16, 16)]
                c = colv[pl.ds(q * 16, 16)]
                col2v[pl.ds(q * 16, 16)] = jnp.where(r == c, n_trash, c)
            pltpu.async_copy(table_hbm.at[rowv], rowsv, sem).wait()
            pltpu.sync_copy(rowsv, shared.at[col2v], add=True)
            return carry

        lax.fori_loop(0, nchunks, chunk, 0)
        plsc.subcore_barrier()
        pltpu.sync_copy(shared.at[pl.ds(sid * rps, rps)],
                        out_hbm.at[cid, pl.ds(sid * rps, rps)])

    return pl.kernel(
        body,
        out_type=jax.ShapeDtypeStruct((_NC, n2, 32), jnp.float32),
        mesh=mesh,
        compiler_params=pltpu.CompilerParams(use_tc_tiling_on_sc=False),
        scratch_types=[
            pltpu.VMEM((_CH,), jnp.int32),
            pltpu.VMEM((_CH,), jnp.int32),
            pltpu.VMEM((_CH,), jnp.int32),
            pltpu.VMEM((_CH, 32), jnp.float32),
            pltpu.VMEM_SHARED((n2, 32), jnp.float32),
            pltpu.SemaphoreType.DMA,
        ],
    )


def _dis_body(p_ref, o_ref):
    deg = 1.0 + p_ref[0, :, 0:1] + p_ref[1, :, 0:1]
    o_ref[...] = lax.rsqrt(deg)


def _hd_body(x_ref, w_ref, dis_ref, o_ref):
    o_ref[...] = dis_ref[...] * jnp.dot(
        x_ref[...], w_ref[...], preferred_element_type=jnp.float32)


def _post_body(p_ref, hd_ref, dis_ref, b_ref, o_ref):
    s = p_ref[0] + p_ref[1] + hd_ref[...]
    o_ref[...] = jnp.tanh(dis_ref[...] * s + b_ref[...])


def _read_body(xc_ref, bt_ref, lv_ref, c5w_ref, c5b_ref, c6w_ref, c6b_ref,
               f1w_ref, f1b_ref, f2w_ref, f2b_ref, o_ref):
    g = pl.program_id(0)
    bt = bt_ref[...]
    mask = bt == g
    cnt = jnp.sum(mask.astype(jnp.int32))
    v = jnp.where(mask, lv_ref[...], _NEG)
    iota = lax.broadcasted_iota(jnp.int32, v.shape, 1)
    big = jnp.int32(2 ** 30)
    ohs = []
    for j in range(30):
        m = jnp.max(v)
        idx = jnp.min(jnp.where(v == m, iota, big))
        oh = iota == idx
        valid = jnp.where(j < cnt, 1.0, 0.0)
        ohs.append(oh.astype(jnp.float32) * valid)
        v = jnp.where(oh, _NEG, v)
    ohm = jnp.concatenate(ohs, axis=0)
    kept = jnp.dot(ohm, xc_ref[...], preferred_element_type=jnp.float32)
    h1 = jnp.maximum(
        jnp.dot(kept, c5w_ref[...], preferred_element_type=jnp.float32)
        + c5b_ref[...], 0.0)
    pool = jnp.concatenate(
        [jnp.maximum(h1[2 * t:2 * t + 1, :], h1[2 * t + 1:2 * t + 2, :])
         for t in range(15)], axis=0)
    acc = None
    for dt in range(5):
        part = jnp.dot(pool[dt:dt + 11, :], c6w_ref[dt],
                       preferred_element_type=jnp.float32)
        acc = part if acc is None else acc + part
    h2 = jnp.maximum(acc + c6b_ref[...], 0.0)
    f = None
    for t in range(11):
        ft = jnp.dot(h2[t:t + 1, :], f1w_ref[t],
                     preferred_element_type=jnp.float32)
        f = ft if f is None else f + ft
    f = jnp.maximum(f + f1b_ref[...], 0.0)
    lg = jnp.dot(f, f2w_ref[...], preferred_element_type=jnp.float32) \
        + f2b_ref[...]
    m2 = jnp.max(lg, axis=1, keepdims=True)
    lg = lg - m2
    ls = lg - jnp.log(jnp.sum(jnp.exp(lg), axis=1, keepdims=True))
    o_ref[...] = jnp.broadcast_to(ls[:, None, :], o_ref.shape)


def _full(shape):
    nd = len(shape)
    return pl.BlockSpec(shape, lambda g, _n=nd: (0,) * _n)


def kernel(x, edge_index, batch, W1, b1, W2, b2, W3, b3, W4, b4,
           c5w, c5b, c6w, c6b, f1w, f1b, f2w, f2b):
    N, F = x.shape
    E = edge_index.shape[1]
    B = 100
    C = f2b.shape[0]
    n2 = ((N + 16 + 127) // 128) * 128
    np3 = ((n2 + 127) // 128) * 128
    epad = ((E + _NW * _CH - 1) // (_NW * _CH)) * (_NW * _CH)

    row = edge_index[0]
    col = edge_index[1]
    if epad > E:
        zpad = jnp.zeros((epad - E,), row.dtype)
        row = jnp.concatenate([row, zpad])
        col = jnp.concatenate([col, zpad])
    row = row.astype(jnp.int32)
    col = col.astype(jnp.int32)

    zeros_tab = jnp.zeros((n2, 32), jnp.float32)
    ones_tab = jnp.ones((n2, 32), jnp.float32)
    sc = _make_sc_scatter(n2, epad)

    p_deg = sc(row, col, ones_tab, zeros_tab)
    dis = pl.pallas_call(
        _dis_body,
        out_shape=jax.ShapeDtypeStruct((n2, 1), jnp.float32),
    )(p_deg)

    def layer(h_in, W, b, width):
        hd = pl.pallas_call(
            _hd_body,
            out_shape=jax.ShapeDtypeStruct((n2, width), jnp.float32),
        )(h_in, W, dis)
        hd32 = hd if width == 32 else jnp.pad(hd, ((0, 0), (0, 32 - width)))
        p = sc(row, col, hd32, zeros_tab)
        pc = p if width == 32 else p[:, :, :width]
        return pl.pallas_call(
            _post_body,
            out_shape=jax.ShapeDtypeStruct((n2, width), jnp.float32),
        )(pc, hd, dis, b.reshape(1, width))

    xp = jnp.pad(x, ((0, n2 - N), (0, 0)))
    x1 = layer(xp, W1, b1, 32)
    x2 = layer(x1, W2, b2, 32)
    x3 = layer(x2, W3, b3, 32)
    x4 = layer(x3, W4, b4, 1)

    xc = jnp.concatenate([x1, x2, x3, x4], axis=-1)
    D = xc.shape[1]
    xcp = jnp.pad(xc, ((0, np3 - n2), (0, 128 - D)))
    lastv = jnp.pad(x4[:, 0], (0, np3 - n2),
                    constant_values=_NEG).reshape(1, np3)
    batchv = jnp.pad(batch.astype(jnp.int32), (0, np3 - N),
                     constant_values=B).reshape(1, np3)

    c5p = jnp.pad(c5w.reshape(16, D).T, ((0, 128 - D), (0, 0)))
    c6p = jnp.transpose(c6w, (2, 1, 0))
    f1p = jnp.transpose(f1w.reshape(32, 11, 128), (1, 0, 2))

    out = pl.pallas_call(
        _read_body,
        grid=(B,),
        in_specs=[
            _full((np3, 128)), _full((1, np3)), _full((1, np3)),
            _full((128, 16)), _full((1, 16)),
            _full((5, 16, 32)), _full((1, 32)),
            _full((11, 32, 128)), _full((1, 128)),
            _full((128, C)), _full((1, C)),
        ],
        out_specs=pl.BlockSpec((1, 8, C), lambda g: (g, 0, 0)),
        out_shape=jax.ShapeDtypeStruct((B, 8, C), jnp.float32),
    )(xcp, batchv, lastv, c5p, c5b.reshape(1, 16), c6p, c6b.reshape(1, 32),
      f1p, f1b.reshape(1, 128), f2w, f2b.reshape(1, C))
    return out[:, 0, :]
